# baseline (speedup 1.0000x reference)
import functools

import jax
import jax.numpy as jnp
from jax import lax
from jax.experimental import pallas as pl
from jax.experimental.pallas import tpu as pltpu
from jax.experimental.pallas import tpu_sc as plsc

_EMB = 64
_NDEP = 46
_NEG = 5
_NW = 32
_CHUNK = 512
_BLK = 512


def _make_sc_gather(B):
    mesh = plsc.VectorSubcoreMesh(core_axis_name="c", subcore_axis_name="s")
    nchunks = {"u": B // _NW // _CHUNK, "o": B // _NW // _CHUNK,
               "n": _NEG * B // _NW // _CHUNK}

    def body(u_hbm, v_hbm, uidx_hbm, oidx_hbm, nidx_hbm,
             uout_hbm, oout_hbm, nout_hbm, idx_v, idx2_v, rows_v, sem):
        wid = lax.axis_index("s") * 2 + lax.axis_index("c")

        def run(idx_hbm, table_hbm, out_hbm, tag):
            nc = nchunks[tag]
            base = wid * nc * _CHUNK
            for c in range(nc):
                off = base + c * _CHUNK
                pltpu.sync_copy(idx_hbm.at[pl.ds(off, _CHUNK)], idx_v)
                for k in range(_CHUNK // 16):
                    idx2_v[pl.ds(k * 16, 16)] = idx_v[pl.ds(k * 16, 16)] >> 1
                pltpu.async_copy(table_hbm.at[idx2_v], rows_v, sem).wait()
                pltpu.sync_copy(rows_v, out_hbm.at[pl.ds(off, _CHUNK)])

        run(uidx_hbm, u_hbm, uout_hbm, "u")
        run(oidx_hbm, v_hbm, oout_hbm, "o")
        run(nidx_hbm, v_hbm, nout_hbm, "n")

    return pl.kernel(
        body,
        mesh=mesh,
        compiler_params=pltpu.CompilerParams(use_tc_tiling_on_sc=True),
        out_type=[jax.ShapeDtypeStruct((B, 2 * _EMB), jnp.float32),
                  jax.ShapeDtypeStruct((B, 2 * _EMB), jnp.float32),
                  jax.ShapeDtypeStruct((_NEG * B, 2 * _EMB), jnp.float32)],
        scratch_types=[pltpu.VMEM((_CHUNK,), jnp.int32),
                       pltpu.VMEM((_CHUNK,), jnp.int32),
                       pltpu.VMEM((_CHUNK, 2 * _EMB), jnp.float32),
                       pltpu.SemaphoreType.DMA],
    )


def _logsig(x):
    return jnp.minimum(x, 0.0) - jnp.log(1.0 + jnp.exp(-jnp.abs(x)))


def _half(pair, idx_col):
    odd = (idx_col & 1) == 1
    return jnp.where(odd, pair[:, _EMB:2 * _EMB], pair[:, 0:_EMB])


def _tc_body(dep_ref, il_ref, ol_ref, ug_ref, uw_ref, ow_ref, nz_ref,
             w_ref, out_ref):
    blk = uw_ref.shape[0]
    uw = _half(uw_ref[...], il_ref[...])
    p = jnp.dot(uw, w_ref[...], preferred_element_type=jnp.float32)
    kid = lax.broadcasted_iota(jnp.int32, (blk, _NDEP * _EMB), 1) >> 6
    masked = jnp.where(kid == dep_ref[...], p, 0.0)
    tin = masked[:, 0:_EMB]
    for k in range(1, _NDEP):
        tin = tin + masked[:, k * _EMB:(k + 1) * _EMB]
    ow = _half(ow_ref[...], ol_ref[...])
    vec_dot = jnp.sum(tin * ow, axis=1, keepdims=True)
    total = jnp.sum(_logsig(vec_dot))
    nsel = _half(nz_ref[...], ug_ref[...])
    tin5 = jnp.repeat(tin, _NEG, axis=0)
    dn = jnp.sum(tin5 * nsel, axis=1, keepdims=True)
    total = total + jnp.sum(_logsig(-dn))

    @pl.when(pl.program_id(0) == 0)
    def _init():
        out_ref[0, 0] = 0.0

    out_ref[0, 0] += total


def _tc_loss(dep2, il2, ol2, ug2, uw2, ow2, nz2, wcols):
    B = uw2.shape[0]
    grid = B // _BLK
    return pl.pallas_call(
        _tc_body,
        grid=(grid,),
        in_specs=[
            pl.BlockSpec((_BLK, 1), lambda i: (i, 0)),
            pl.BlockSpec((_BLK, 1), lambda i: (i, 0)),
            pl.BlockSpec((_BLK, 1), lambda i: (i, 0)),
            pl.BlockSpec((_NEG * _BLK, 1), lambda i: (i, 0)),
            pl.BlockSpec((_BLK, 2 * _EMB), lambda i: (i, 0)),
            pl.BlockSpec((_BLK, 2 * _EMB), lambda i: (i, 0)),
            pl.BlockSpec((_NEG * _BLK, 2 * _EMB), lambda i: (i, 0)),
            pl.BlockSpec((_EMB, _NDEP * _EMB), lambda i: (0, 0)),
        ],
        out_specs=pl.BlockSpec(memory_space=pltpu.MemorySpace.SMEM),
        out_shape=jax.ShapeDtypeStruct((1, 1), jnp.float32),
    )(dep2, il2, ol2, ug2, uw2, ow2, nz2, wcols)


def kernel(input_label, out_label, dep_label, use_given, u_emb, v_emb,
           dep_mxs):
    B = out_label.shape[0]
    V = u_emb.shape[0]
    u2 = u_emb.reshape(V // 2, 2 * _EMB)
    v2 = v_emb.reshape(V // 2, 2 * _EMB)
    nidx = use_given.reshape(-1)
    uw2, ow2, nz2 = _make_sc_gather(B)(u2, v2, input_label, out_label, nidx)
    wcols = jnp.transpose(dep_mxs.reshape(_NDEP, _EMB, _EMB),
                          (1, 0, 2)).reshape(_EMB, _NDEP * _EMB)
    res = _tc_loss(dep_label.reshape(B, 1), input_label.reshape(B, 1),
                   out_label.reshape(B, 1), nidx.reshape(_NEG * B, 1),
                   uw2, ow2, nz2, wcols)
    return -res[0, 0] / B


# speedup vs baseline: 1.1263x; 1.1263x over previous
import functools

import jax
import jax.numpy as jnp
from jax import lax
from jax.experimental import pallas as pl
from jax.experimental.pallas import tpu as pltpu
from jax.experimental.pallas import tpu_sc as plsc

_EMB = 64
_NDEP = 46
_NEG = 5
_NW = 32
_CHUNK = 512
_BLK = 512


def _make_sc_gather(B):
    mesh = plsc.VectorSubcoreMesh(core_axis_name="c", subcore_axis_name="s")

    def body(u_hbm, v_hbm, uidx_hbm, oidx_hbm, nidx_hbm,
             uout_hbm, oout_hbm, n0_hbm, n1_hbm, n2_hbm, n3_hbm, n4_hbm,
             idx_v, rows_v, sem):
        wid = lax.axis_index("s") * 2 + lax.axis_index("c")
        nouts = [n0_hbm, n1_hbm, n2_hbm, n3_hbm, n4_hbm]

        def chunk(idx_hbm, idx_off, table_hbm, out_hbm, out_off):
            pltpu.sync_copy(idx_hbm.at[pl.ds(idx_off, _CHUNK)], idx_v)
            pltpu.async_copy(table_hbm.at[idx_v], rows_v, sem).wait()
            pltpu.sync_copy(rows_v, out_hbm.at[pl.ds(out_off, _CHUNK)])

        nc = B // _NW // _CHUNK
        for c in range(nc):
            off = wid * nc * _CHUNK + c * _CHUNK
            chunk(uidx_hbm, off, u_hbm, uout_hbm, off)
        for c in range(nc):
            off = wid * nc * _CHUNK + c * _CHUNK
            chunk(oidx_hbm, off, v_hbm, oout_hbm, off)
        for n in range(_NEG):
            for c in range(nc):
                off = wid * nc * _CHUNK + c * _CHUNK
                chunk(nidx_hbm, n * B + off, v_hbm, nouts[n], off)

    return pl.kernel(
        body,
        mesh=mesh,
        compiler_params=pltpu.CompilerParams(use_tc_tiling_on_sc=False),
        out_type=[jax.ShapeDtypeStruct((B, _EMB), jnp.float32)
                  for _ in range(2 + _NEG)],
        scratch_types=[pltpu.VMEM((_CHUNK,), jnp.int32),
                       pltpu.VMEM((_CHUNK, _EMB), jnp.float32),
                       pltpu.SemaphoreType.DMA],
    )


def _logsig(x):
    return jnp.minimum(x, 0.0) - jnp.log(1.0 + jnp.exp(-jnp.abs(x)))


def _tc_body(dep_ref, uw_ref, ow_ref, n0_ref, n1_ref, n2_ref, n3_ref,
             n4_ref, w_ref, out_ref):
    blk = uw_ref.shape[0]
    p = jnp.dot(uw_ref[...], w_ref[...], preferred_element_type=jnp.float32)
    kid = lax.broadcasted_iota(jnp.int32, (blk, _NDEP * _EMB), 1) >> 6
    masked = jnp.where(kid == dep_ref[...], p, 0.0)
    tin = masked[:, 0:_EMB]
    for k in range(1, _NDEP):
        tin = tin + masked[:, k * _EMB:(k + 1) * _EMB]
    prods = jnp.concatenate(
        [tin * ow_ref[...], tin * n0_ref[...], tin * n1_ref[...],
         tin * n2_ref[...], tin * n3_ref[...], tin * n4_ref[...]], axis=1)
    rseg = lax.broadcasted_iota(jnp.int32, (6 * _EMB, 6), 0) >> 6
    cseg = lax.broadcasted_iota(jnp.int32, (6 * _EMB, 6), 1)
    ones_bd = jnp.where(rseg == cseg, 1.0, 0.0)
    dots = jnp.dot(prods, ones_bd, preferred_element_type=jnp.float32)
    csign = lax.broadcasted_iota(jnp.int32, (blk, 6), 1)
    x = jnp.where(csign == 0, dots, -dots)
    total = jnp.sum(_logsig(x))

    @pl.when(pl.program_id(0) == 0)
    def _init():
        out_ref[0, 0] = 0.0

    out_ref[0, 0] += total


def _tc_loss(dep2, uw, ow, nzs, wcols):
    B = uw.shape[0]
    grid = B // _BLK
    bspec = pl.BlockSpec((_BLK, _EMB), lambda i: (i, 0))
    return pl.pallas_call(
        _tc_body,
        grid=(grid,),
        in_specs=[pl.BlockSpec((_BLK, 1), lambda i: (i, 0)),
                  bspec, bspec, bspec, bspec, bspec, bspec, bspec,
                  pl.BlockSpec((_EMB, _NDEP * _EMB), lambda i: (0, 0))],
        out_specs=pl.BlockSpec(memory_space=pltpu.MemorySpace.SMEM),
        out_shape=jax.ShapeDtypeStruct((1, 1), jnp.float32),
    )(dep2, uw, ow, *nzs, wcols)


def kernel(input_label, out_label, dep_label, use_given, u_emb, v_emb,
           dep_mxs):
    B = out_label.shape[0]
    nidx = use_given.T.reshape(-1)
    uw, ow, n0, n1, n2, n3, n4 = _make_sc_gather(B)(
        u_emb, v_emb, input_label, out_label, nidx)
    wcols = jnp.transpose(dep_mxs.reshape(_NDEP, _EMB, _EMB),
                          (1, 0, 2)).reshape(_EMB, _NDEP * _EMB)
    res = _tc_loss(dep_label.reshape(B, 1), uw, ow, (n0, n1, n2, n3, n4),
                   wcols)
    return -res[0, 0] / B


# speedup vs baseline: 1.4132x; 1.2547x over previous
import functools

import jax
import jax.numpy as jnp
from jax import lax
from jax.experimental import pallas as pl
from jax.experimental.pallas import tpu as pltpu
from jax.experimental.pallas import tpu_sc as plsc

_EMB = 64
_NDEP = 46
_NEG = 5
_NW = 32
_CHUNK = 512
_BLK = 512
_VC = 16384


def _retile_body(t_ref, out_ref):
    t = jnp.transpose(t_ref[...], (1, 0))
    t3 = t.reshape(_VC // 2, 2, _EMB)
    out_ref[...] = jnp.concatenate([t3[:, 0, :], t3[:, 1, :]], axis=1)


def _retile(tt, V):
    grid = (V + _VC - 1) // _VC
    return pl.pallas_call(
        _retile_body,
        grid=(grid,),
        in_specs=[pl.BlockSpec((_EMB, _VC), lambda i: (0, i))],
        out_specs=pl.BlockSpec((_VC // 2, 2 * _EMB), lambda i: (i, 0)),
        out_shape=jax.ShapeDtypeStruct((V // 2, 2 * _EMB), jnp.float32),
    )(tt)


def _make_sc_gather(B):
    mesh = plsc.VectorSubcoreMesh(core_axis_name="c", subcore_axis_name="s")

    def body(u_hbm, v_hbm, uidx_hbm, oidx_hbm, nidx_hbm,
             uout_hbm, oout_hbm, n0_hbm, n1_hbm, n2_hbm, n3_hbm, n4_hbm,
             idx_v, idx2_v, rows_v, sem):
        wid = lax.axis_index("s") * 2 + lax.axis_index("c")
        nouts = [n0_hbm, n1_hbm, n2_hbm, n3_hbm, n4_hbm]

        def chunk(idx_hbm, idx_off, table_hbm, out_hbm, out_off):
            pltpu.sync_copy(idx_hbm.at[pl.ds(idx_off, _CHUNK)], idx_v)
            for k in range(_CHUNK // 16):
                idx2_v[pl.ds(k * 16, 16)] = idx_v[pl.ds(k * 16, 16)] >> 1
            pltpu.async_copy(table_hbm.at[idx2_v], rows_v, sem).wait()
            pltpu.sync_copy(rows_v, out_hbm.at[pl.ds(out_off, _CHUNK)])

        nc = B // _NW // _CHUNK
        for c in range(nc):
            off = wid * nc * _CHUNK + c * _CHUNK
            chunk(uidx_hbm, off, u_hbm, uout_hbm, off)
        for c in range(nc):
            off = wid * nc * _CHUNK + c * _CHUNK
            chunk(oidx_hbm, off, v_hbm, oout_hbm, off)
        for n in range(_NEG):
            for c in range(nc):
                off = wid * nc * _CHUNK + c * _CHUNK
                chunk(nidx_hbm, n * B + off, v_hbm, nouts[n], off)

    return pl.kernel(
        body,
        mesh=mesh,
        compiler_params=pltpu.CompilerParams(use_tc_tiling_on_sc=True),
        out_type=[jax.ShapeDtypeStruct((B, 2 * _EMB), jnp.float32)
                  for _ in range(2 + _NEG)],
        scratch_types=[pltpu.VMEM((_CHUNK,), jnp.int32),
                       pltpu.VMEM((_CHUNK,), jnp.int32),
                       pltpu.VMEM((_CHUNK, 2 * _EMB), jnp.float32),
                       pltpu.SemaphoreType.DMA],
    )


def _logsig(x):
    return jnp.minimum(x, 0.0) - jnp.log(1.0 + jnp.exp(-jnp.abs(x)))


def _half(pair, idx_col):
    odd = (idx_col & 1) == 1
    return jnp.where(odd, pair[:, _EMB:2 * _EMB], pair[:, 0:_EMB])


def _tc_body(dep_ref, il_ref, ol_ref, g0_ref, g1_ref, g2_ref, g3_ref,
             g4_ref, uw_ref, ow_ref, n0_ref, n1_ref, n2_ref, n3_ref,
             n4_ref, w_ref, out_ref):
    blk = uw_ref.shape[0]
    uw = _half(uw_ref[...], il_ref[...])
    p = jnp.dot(uw, w_ref[...], preferred_element_type=jnp.float32)
    kid = lax.broadcasted_iota(jnp.int32, (blk, _NDEP * _EMB), 1) >> 6
    masked = jnp.where(kid == dep_ref[...], p, 0.0)
    tin = masked[:, 0:_EMB]
    for k in range(1, _NDEP):
        tin = tin + masked[:, k * _EMB:(k + 1) * _EMB]
    ow = _half(ow_ref[...], ol_ref[...])
    nz = [_half(n_ref[...], g_ref[...])
          for n_ref, g_ref in ((n0_ref, g0_ref), (n1_ref, g1_ref),
                               (n2_ref, g2_ref), (n3_ref, g3_ref),
                               (n4_ref, g4_ref))]
    prods = jnp.concatenate([tin * ow] + [tin * z for z in nz], axis=1)
    rseg = lax.broadcasted_iota(jnp.int32, (6 * _EMB, 6), 0) >> 6
    cseg = lax.broadcasted_iota(jnp.int32, (6 * _EMB, 6), 1)
    ones_bd = jnp.where(rseg == cseg, 1.0, 0.0)
    dots = jnp.dot(prods, ones_bd, preferred_element_type=jnp.float32)
    csign = lax.broadcasted_iota(jnp.int32, (blk, 6), 1)
    x = jnp.where(csign == 0, dots, -dots)
    total = jnp.sum(_logsig(x))

    @pl.when(pl.program_id(0) == 0)
    def _init():
        out_ref[0, 0] = 0.0

    out_ref[0, 0] += total


def _tc_loss(dep2, il2, ol2, gs, uw, ow, nzs, wcols):
    B = uw.shape[0]
    grid = B // _BLK
    ispec = pl.BlockSpec((_BLK, 1), lambda i: (i, 0))
    bspec = pl.BlockSpec((_BLK, 2 * _EMB), lambda i: (i, 0))
    return pl.pallas_call(
        _tc_body,
        grid=(grid,),
        in_specs=[ispec, ispec, ispec, ispec, ispec, ispec, ispec, ispec,
                  bspec, bspec, bspec, bspec, bspec, bspec, bspec,
                  pl.BlockSpec((_EMB, _NDEP * _EMB), lambda i: (0, 0))],
        out_specs=pl.BlockSpec(memory_space=pltpu.MemorySpace.SMEM),
        out_shape=jax.ShapeDtypeStruct((1, 1), jnp.float32),
    )(dep2, il2, ol2, *gs, uw, ow, *nzs, wcols)


def kernel(input_label, out_label, dep_label, use_given, u_emb, v_emb,
           dep_mxs):
    B = out_label.shape[0]
    V = u_emb.shape[0]
    u2 = _retile(u_emb.T, V)
    v2 = _retile(v_emb.T, V)
    nidx = use_given.T.reshape(-1)
    uw, ow, n0, n1, n2, n3, n4 = _make_sc_gather(B)(
        u2, v2, input_label, out_label, nidx)
    wcols = jnp.transpose(dep_mxs.reshape(_NDEP, _EMB, _EMB),
                          (1, 0, 2)).reshape(_EMB, _NDEP * _EMB)
    gs = tuple(use_given[:, n].reshape(B, 1) for n in range(_NEG))
    res = _tc_loss(dep_label.reshape(B, 1), input_label.reshape(B, 1),
                   out_label.reshape(B, 1), gs, uw, ow,
                   (n0, n1, n2, n3, n4), wcols)
    return -res[0, 0] / B


# speedup vs baseline: 1.4997x; 1.0612x over previous
import functools

import jax
import jax.numpy as jnp
from jax import lax
from jax.experimental import pallas as pl
from jax.experimental.pallas import tpu as pltpu
from jax.experimental.pallas import tpu_sc as plsc

_EMB = 64
_NDEP = 46
_NEG = 5
_NW = 32
_CHUNK = 512
_BLK = 512
_VC = 16384


def _retile_body(t_ref, out_ref):
    t = jnp.transpose(t_ref[...], (1, 0))
    t3 = t.reshape(_VC // 2, 2, _EMB)
    out_ref[...] = jnp.concatenate([t3[:, 0, :], t3[:, 1, :]], axis=1)


def _retile(tt, V):
    grid = (V + _VC - 1) // _VC
    return pl.pallas_call(
        _retile_body,
        grid=(grid,),
        in_specs=[pl.BlockSpec((_EMB, _VC), lambda i: (0, i))],
        out_specs=pl.BlockSpec((_VC // 2, 2 * _EMB), lambda i: (i, 0)),
        out_shape=jax.ShapeDtypeStruct((V // 2, 2 * _EMB), jnp.float32),
    )(tt)


def _sc_chunk(idx_hbm, idx_off, table_hbm, out_hbm, out_off,
              idx_v, idx2_v, rows_v, sem):
    pltpu.sync_copy(idx_hbm.at[pl.ds(idx_off, _CHUNK)], idx_v)
    for k in range(_CHUNK // 16):
        idx2_v[pl.ds(k * 16, 16)] = idx_v[pl.ds(k * 16, 16)] >> 1
    pltpu.async_copy(table_hbm.at[idx2_v], rows_v, sem).wait()
    pltpu.sync_copy(rows_v, out_hbm.at[pl.ds(out_off, _CHUNK)])


def _sc_scratch():
    return [pltpu.VMEM((_CHUNK,), jnp.int32),
            pltpu.VMEM((_CHUNK,), jnp.int32),
            pltpu.VMEM((_CHUNK, 2 * _EMB), jnp.float32),
            pltpu.SemaphoreType.DMA]


def _sc_mesh():
    return plsc.VectorSubcoreMesh(core_axis_name="c", subcore_axis_name="s")


def _make_sc_gather_v(B):
    def body(v_hbm, oidx_hbm, nidx_hbm,
             oout_hbm, n0_hbm, n1_hbm, n2_hbm, n3_hbm, n4_hbm,
             idx_v, idx2_v, rows_v, sem):
        wid = lax.axis_index("s") * 2 + lax.axis_index("c")
        nouts = [n0_hbm, n1_hbm, n2_hbm, n3_hbm, n4_hbm]
        nc = B // _NW // _CHUNK
        for c in range(nc):
            off = wid * nc * _CHUNK + c * _CHUNK
            _sc_chunk(oidx_hbm, off, v_hbm, oout_hbm, off,
                      idx_v, idx2_v, rows_v, sem)
        for n in range(_NEG):
            for c in range(nc):
                off = wid * nc * _CHUNK + c * _CHUNK
                _sc_chunk(nidx_hbm, n * B + off, v_hbm, nouts[n], off,
                          idx_v, idx2_v, rows_v, sem)

    return pl.kernel(
        body,
        mesh=_sc_mesh(),
        compiler_params=pltpu.CompilerParams(use_tc_tiling_on_sc=True),
        out_type=[jax.ShapeDtypeStruct((B, 2 * _EMB), jnp.float32)
                  for _ in range(1 + _NEG)],
        scratch_types=_sc_scratch(),
    )


def _make_sc_gather_u(B):
    def body(u_hbm, uidx_hbm, uout_hbm, idx_v, idx2_v, rows_v, sem):
        wid = lax.axis_index("s") * 2 + lax.axis_index("c")
        nc = B // _NW // _CHUNK
        for c in range(nc):
            off = wid * nc * _CHUNK + c * _CHUNK
            _sc_chunk(uidx_hbm, off, u_hbm, uout_hbm, off,
                      idx_v, idx2_v, rows_v, sem)

    return pl.kernel(
        body,
        mesh=_sc_mesh(),
        compiler_params=pltpu.CompilerParams(use_tc_tiling_on_sc=True),
        out_type=[jax.ShapeDtypeStruct((B, 2 * _EMB), jnp.float32)],
        scratch_types=_sc_scratch(),
    )


def _logsig(x):
    return jnp.minimum(x, 0.0) - jnp.log(1.0 + jnp.exp(-jnp.abs(x)))


def _half(pair, idx_col):
    odd = (idx_col & 1) == 1
    return jnp.where(odd, pair[:, _EMB:2 * _EMB], pair[:, 0:_EMB])


def _tc_body(ids_ref, uw_ref, ow_ref, n0_ref, n1_ref, n2_ref, n3_ref,
             n4_ref, w_ref, out_ref):
    blk = uw_ref.shape[0]
    uw = _half(uw_ref[...], ids_ref[:, 1:2])
    p = jnp.dot(uw, w_ref[...], preferred_element_type=jnp.float32)
    kid = lax.broadcasted_iota(jnp.int32, (blk, _NDEP * _EMB), 1) >> 6
    masked = jnp.where(kid == ids_ref[:, 0:1], p, 0.0)
    tin = masked[:, 0:_EMB]
    for k in range(1, _NDEP):
        tin = tin + masked[:, k * _EMB:(k + 1) * _EMB]
    ow = _half(ow_ref[...], ids_ref[:, 2:3])
    nz = [_half(n_ref[...], ids_ref[:, 3 + n:4 + n])
          for n, n_ref in enumerate((n0_ref, n1_ref, n2_ref, n3_ref,
                                     n4_ref))]
    prods = jnp.concatenate([tin * ow] + [tin * z for z in nz], axis=1)
    rseg = lax.broadcasted_iota(jnp.int32, (6 * _EMB, 6), 0) >> 6
    cseg = lax.broadcasted_iota(jnp.int32, (6 * _EMB, 6), 1)
    ones_bd = jnp.where(rseg == cseg, 1.0, 0.0)
    dots = jnp.dot(prods, ones_bd, preferred_element_type=jnp.float32)
    csign = lax.broadcasted_iota(jnp.int32, (blk, 6), 1)
    x = jnp.where(csign == 0, dots, -dots)
    total = jnp.sum(_logsig(x))

    @pl.when(pl.program_id(0) == 0)
    def _init():
        out_ref[0, 0] = 0.0

    out_ref[0, 0] += total


def _tc_loss(ids, uw, ow, nzs, wcols):
    B = uw.shape[0]
    grid = B // _BLK
    bspec = pl.BlockSpec((_BLK, 2 * _EMB), lambda i: (i, 0))
    return pl.pallas_call(
        _tc_body,
        grid=(grid,),
        in_specs=[pl.BlockSpec((_BLK, 8), lambda i: (i, 0)),
                  bspec, bspec, bspec, bspec, bspec, bspec, bspec,
                  pl.BlockSpec((_EMB, _NDEP * _EMB), lambda i: (0, 0))],
        out_specs=pl.BlockSpec(memory_space=pltpu.MemorySpace.SMEM),
        out_shape=jax.ShapeDtypeStruct((1, 1), jnp.float32),
    )(ids, uw, ow, *nzs, wcols)


def kernel(input_label, out_label, dep_label, use_given, u_emb, v_emb,
           dep_mxs):
    B = out_label.shape[0]
    V = u_emb.shape[0]
    v2 = _retile(v_emb.T, V)
    nidx = use_given.T.reshape(-1)
    ow, n0, n1, n2, n3, n4 = _make_sc_gather_v(B)(v2, out_label, nidx)
    u2 = _retile(u_emb.T, V)
    (uw,) = _make_sc_gather_u(B)(u2, input_label)
    wcols = jnp.transpose(dep_mxs.reshape(_NDEP, _EMB, _EMB),
                          (1, 0, 2)).reshape(_EMB, _NDEP * _EMB)
    ids = jnp.concatenate(
        [dep_label.reshape(B, 1), input_label.reshape(B, 1),
         out_label.reshape(B, 1), use_given], axis=1)
    res = _tc_loss(ids, uw, ow, (n0, n1, n2, n3, n4), wcols)
    return -res[0, 0] / B


# speedup vs baseline: 1.5002x; 1.0004x over previous
import functools

import jax
import jax.numpy as jnp
from jax import lax
from jax.experimental import pallas as pl
from jax.experimental.pallas import tpu as pltpu
from jax.experimental.pallas import tpu_sc as plsc

_EMB = 64
_NDEP = 46
_NEG = 5
_NW = 32
_CHUNK = 512
_BLK = 2048
_VC = 16384


def _retile_body(t_ref, out_ref):
    t = jnp.transpose(t_ref[...], (1, 0))
    t3 = t.reshape(_VC // 2, 2, _EMB)
    out_ref[...] = jnp.concatenate([t3[:, 0, :], t3[:, 1, :]], axis=1)


def _retile(tt, V):
    grid = (V + _VC - 1) // _VC
    return pl.pallas_call(
        _retile_body,
        grid=(grid,),
        in_specs=[pl.BlockSpec((_EMB, _VC), lambda i: (0, i))],
        out_specs=pl.BlockSpec((_VC // 2, 2 * _EMB), lambda i: (i, 0)),
        out_shape=jax.ShapeDtypeStruct((V // 2, 2 * _EMB), jnp.float32),
    )(tt)


def _sc_chunk(idx_hbm, idx_off, table_hbm, out_hbm, out_off,
              idx_v, idx2_v, rows_v, sem):
    pltpu.sync_copy(idx_hbm.at[pl.ds(idx_off, _CHUNK)], idx_v)
    for k in range(_CHUNK // 16):
        idx2_v[pl.ds(k * 16, 16)] = idx_v[pl.ds(k * 16, 16)] >> 1
    pltpu.async_copy(table_hbm.at[idx2_v], rows_v, sem).wait()
    pltpu.sync_copy(rows_v, out_hbm.at[pl.ds(out_off, _CHUNK)])


def _sc_scratch():
    return [pltpu.VMEM((_CHUNK,), jnp.int32),
            pltpu.VMEM((_CHUNK,), jnp.int32),
            pltpu.VMEM((_CHUNK, 2 * _EMB), jnp.float32),
            pltpu.SemaphoreType.DMA]


def _sc_mesh():
    return plsc.VectorSubcoreMesh(core_axis_name="c", subcore_axis_name="s")


def _make_sc_gather_v(B):
    def body(v_hbm, oidx_hbm, nidx_hbm,
             oout_hbm, n0_hbm, n1_hbm, n2_hbm, n3_hbm, n4_hbm,
             idx_v, idx2_v, rows_v, sem):
        wid = lax.axis_index("s") * 2 + lax.axis_index("c")
        nouts = [n0_hbm, n1_hbm, n2_hbm, n3_hbm, n4_hbm]
        nc = B // _NW // _CHUNK
        for c in range(nc):
            off = wid * nc * _CHUNK + c * _CHUNK
            _sc_chunk(oidx_hbm, off, v_hbm, oout_hbm, off,
                      idx_v, idx2_v, rows_v, sem)
        for n in range(_NEG):
            for c in range(nc):
                off = wid * nc * _CHUNK + c * _CHUNK
                _sc_chunk(nidx_hbm, n * B + off, v_hbm, nouts[n], off,
                          idx_v, idx2_v, rows_v, sem)

    return pl.kernel(
        body,
        mesh=_sc_mesh(),
        compiler_params=pltpu.CompilerParams(use_tc_tiling_on_sc=True),
        out_type=[jax.ShapeDtypeStruct((B, 2 * _EMB), jnp.float32)
                  for _ in range(1 + _NEG)],
        scratch_types=_sc_scratch(),
    )


def _make_sc_gather_u(B):
    def body(u_hbm, uidx_hbm, uout_hbm, idx_v, idx2_v, rows_v, sem):
        wid = lax.axis_index("s") * 2 + lax.axis_index("c")
        nc = B // _NW // _CHUNK
        for c in range(nc):
            off = wid * nc * _CHUNK + c * _CHUNK
            _sc_chunk(uidx_hbm, off, u_hbm, uout_hbm, off,
                      idx_v, idx2_v, rows_v, sem)

    return pl.kernel(
        body,
        mesh=_sc_mesh(),
        compiler_params=pltpu.CompilerParams(use_tc_tiling_on_sc=True),
        out_type=[jax.ShapeDtypeStruct((B, 2 * _EMB), jnp.float32)],
        scratch_types=_sc_scratch(),
    )


def _logsig(x):
    return jnp.minimum(x, 0.0) - jnp.log(1.0 + jnp.exp(-jnp.abs(x)))


def _half(pair, idx_col):
    odd = (idx_col & 1) == 1
    return jnp.where(odd, pair[:, _EMB:2 * _EMB], pair[:, 0:_EMB])


def _tc_body(ids_ref, uw_ref, ow_ref, n0_ref, n1_ref, n2_ref, n3_ref,
             n4_ref, w_ref, out_ref):
    blk = uw_ref.shape[0]
    uw = _half(uw_ref[...], ids_ref[:, 1:2])
    p = jnp.dot(uw, w_ref[...], preferred_element_type=jnp.float32)
    kid = lax.broadcasted_iota(jnp.int32, (blk, _NDEP * _EMB), 1) >> 6
    masked = jnp.where(kid == ids_ref[:, 0:1], p, 0.0)
    tin = masked[:, 0:_EMB]
    for k in range(1, _NDEP):
        tin = tin + masked[:, k * _EMB:(k + 1) * _EMB]
    ow = _half(ow_ref[...], ids_ref[:, 2:3])
    nz = [_half(n_ref[...], ids_ref[:, 3 + n:4 + n])
          for n, n_ref in enumerate((n0_ref, n1_ref, n2_ref, n3_ref,
                                     n4_ref))]
    prods = jnp.concatenate([tin * ow] + [tin * z for z in nz], axis=1)
    rseg = lax.broadcasted_iota(jnp.int32, (6 * _EMB, 6), 0) >> 6
    cseg = lax.broadcasted_iota(jnp.int32, (6 * _EMB, 6), 1)
    ones_bd = jnp.where(rseg == cseg, 1.0, 0.0)
    dots = jnp.dot(prods, ones_bd, preferred_element_type=jnp.float32)
    csign = lax.broadcasted_iota(jnp.int32, (blk, 6), 1)
    x = jnp.where(csign == 0, dots, -dots)
    total = jnp.sum(_logsig(x))

    @pl.when(pl.program_id(0) == 0)
    def _init():
        out_ref[0, 0] = 0.0

    out_ref[0, 0] += total


def _tc_loss(ids, uw, ow, nzs, wcols):
    B = uw.shape[0]
    grid = B // _BLK
    bspec = pl.BlockSpec((_BLK, 2 * _EMB), lambda i: (i, 0))
    return pl.pallas_call(
        _tc_body,
        grid=(grid,),
        in_specs=[pl.BlockSpec((_BLK, 8), lambda i: (i, 0)),
                  bspec, bspec, bspec, bspec, bspec, bspec, bspec,
                  pl.BlockSpec((_EMB, _NDEP * _EMB), lambda i: (0, 0))],
        out_specs=pl.BlockSpec(memory_space=pltpu.MemorySpace.SMEM),
        out_shape=jax.ShapeDtypeStruct((1, 1), jnp.float32),
    )(ids, uw, ow, *nzs, wcols)


def kernel(input_label, out_label, dep_label, use_given, u_emb, v_emb,
           dep_mxs):
    B = out_label.shape[0]
    V = u_emb.shape[0]
    v2 = _retile(v_emb.T, V)
    nidx = use_given.T.reshape(-1)
    ow, n0, n1, n2, n3, n4 = _make_sc_gather_v(B)(v2, out_label, nidx)
    u2 = _retile(u_emb.T, V)
    (uw,) = _make_sc_gather_u(B)(u2, input_label)
    wcols = jnp.transpose(dep_mxs.reshape(_NDEP, _EMB, _EMB),
                          (1, 0, 2)).reshape(_EMB, _NDEP * _EMB)
    ids = jnp.concatenate(
        [dep_label.reshape(B, 1), input_label.reshape(B, 1),
         out_label.reshape(B, 1), use_given], axis=1)
    res = _tc_loss(ids, uw, ow, (n0, n1, n2, n3, n4), wcols)
    return -res[0, 0] / B
